# Initial kernel scaffold; baseline (speedup 1.0000x reference)
import jax
import jax.numpy as jnp
from jax.experimental import pallas as pl


def kernel(x, edge_index, W_rel, b_rel, W_root):
    raise NotImplementedError("write your pallas kernel here")


# speedup vs baseline: 2.8921x; 2.8921x over previous
import functools

import jax
import jax.numpy as jnp
from jax import lax
from jax.experimental import pallas as pl
from jax.experimental.pallas import tpu as pltpu
from jax.experimental.pallas import tpu_sc as plsc

NC = 2
NS = 16
NW = NC * NS
LANE = 128


def _sc_aggregate(x_pad, src2d, dst2d, zeros_hbm, acc_rows, rows_per_worker):
    rows_per_sub = acc_rows // NS
    mesh = plsc.VectorSubcoreMesh(core_axis_name="c", subcore_axis_name="s")

    @functools.partial(
        pl.kernel,
        mesh=mesh,
        out_type=jax.ShapeDtypeStruct((NC, acc_rows, LANE), jnp.float32),
        scratch_types=[
            pltpu.VMEM_SHARED((acc_rows, LANE), jnp.float32),
            pltpu.VMEM((rows_per_worker, LANE), jnp.int32),
            pltpu.VMEM((rows_per_worker, LANE), jnp.int32),
            pltpu.VMEM((LANE, LANE), jnp.float32),
            pltpu.SemaphoreType.DMA,
        ],
    )
    def agg_kernel(x_hbm, src_hbm, dst_hbm, z_hbm, out_hbm,
                   acc, src_v, dst_v, rows_v, sem):
        c = lax.axis_index("c")
        s = lax.axis_index("s")
        wid = c * NS + s

        pltpu.sync_copy(z_hbm, acc.at[pl.ds(s * rows_per_sub, rows_per_sub)])
        plsc.subcore_barrier()

        base = wid * rows_per_worker
        pltpu.sync_copy(src_hbm.at[pl.ds(base, rows_per_worker)], src_v)
        pltpu.sync_copy(dst_hbm.at[pl.ds(base, rows_per_worker)], dst_v)

        def step(j, carry):
            pltpu.async_copy(x_hbm.at[src_v.at[j]], rows_v, sem).wait()
            pltpu.sync_copy(rows_v, acc.at[dst_v.at[j]], add=True)
            return carry

        lax.fori_loop(0, rows_per_worker, step, 0)
        plsc.subcore_barrier()

        pltpu.sync_copy(acc.at[pl.ds(s * rows_per_sub, rows_per_sub)],
                        out_hbm.at[c, pl.ds(s * rows_per_sub, rows_per_sub)])

    return agg_kernel(x_pad, src2d, dst2d, zeros_hbm)


def _tc_combine(partials, x, W_rel, b_rel, W_root):
    n = x.shape[0]
    blk = 80
    grid = n // blk

    def body(p_ref, x_ref, wrel_ref, wroot_ref, b_ref, o_ref):
        agg = p_ref[0] + p_ref[1]
        o_ref[...] = (
            lax.dot_general(agg, wrel_ref[...], (((1,), (1,)), ((), ())),
                            preferred_element_type=jnp.float32)
            + lax.dot_general(x_ref[...], wroot_ref[...], (((1,), (1,)), ((), ())),
                              preferred_element_type=jnp.float32)
            + b_ref[...]
        )

    return pl.pallas_call(
        body,
        grid=(grid,),
        in_specs=[
            pl.BlockSpec((NC, blk, LANE), lambda i: (0, i, 0)),
            pl.BlockSpec((blk, LANE), lambda i: (i, 0)),
            pl.BlockSpec((LANE, LANE), lambda i: (0, 0)),
            pl.BlockSpec((LANE, LANE), lambda i: (0, 0)),
            pl.BlockSpec((1, LANE), lambda i: (0, 0)),
        ],
        out_specs=pl.BlockSpec((blk, LANE), lambda i: (i, 0)),
        out_shape=jax.ShapeDtypeStruct((n, LANE), jnp.float32),
    )(partials, x, W_rel, W_root, b_rel.reshape(1, LANE))


def kernel(x, edge_index, W_rel, b_rel, W_root):
    n, d = x.shape
    e = edge_index.shape[1]
    src = edge_index[0].astype(jnp.int32)
    dst = edge_index[1].astype(jnp.int32)

    per_worker = 8 * LANE * ((e + NW * 8 * LANE - 1) // (NW * 8 * LANE))
    e_pad = NW * per_worker
    src = jnp.concatenate([src, jnp.full((e_pad - e,), n, jnp.int32)])
    dst = jnp.concatenate([dst, jnp.full((e_pad - e,), n, jnp.int32)])
    src2d = src.reshape(-1, LANE)
    dst2d = dst.reshape(-1, LANE)

    x_rows = ((n + 16 + 7) // 8) * 8
    x_pad = jnp.zeros((x_rows, d), jnp.float32).at[:n].set(x)

    acc_rows = ((n + 1 + NS * 8 - 1) // (NS * 8)) * (NS * 8)
    zeros_hbm = jnp.zeros((acc_rows // NS, LANE), jnp.float32)

    partials = _sc_aggregate(x_pad, src2d, dst2d, zeros_hbm,
                             acc_rows, per_worker // LANE)
    return _tc_combine(partials[:, :n], x, W_rel, b_rel, W_root)


# speedup vs baseline: 3.0539x; 1.0560x over previous
import functools

import jax
import jax.numpy as jnp
from jax import lax
from jax.experimental import pallas as pl
from jax.experimental.pallas import tpu as pltpu
from jax.experimental.pallas import tpu_sc as plsc

NC = 2
NS = 16
NW = NC * NS
LANE = 128


def _sc_aggregate(x_pad, src2d, dst2d, zeros_hbm, acc_rows, rows_per_worker):
    rows_per_sub = acc_rows // NS
    half = rows_per_worker // 2
    mesh = plsc.VectorSubcoreMesh(core_axis_name="c", subcore_axis_name="s")

    @functools.partial(
        pl.kernel,
        mesh=mesh,
        out_type=jax.ShapeDtypeStruct((NC, acc_rows, LANE), jnp.float32),
        scratch_types=[
            pltpu.VMEM_SHARED((acc_rows, LANE), jnp.float32),
            pltpu.VMEM((half, LANE), jnp.int32),
            pltpu.VMEM((half, LANE), jnp.int32),
            pltpu.VMEM((LANE, LANE), jnp.float32),
            pltpu.VMEM((LANE, LANE), jnp.float32),
            pltpu.SemaphoreType.DMA,
            pltpu.SemaphoreType.DMA,
        ],
    )
    def agg_kernel(x_hbm, src_hbm, dst_hbm, z_hbm, out_hbm,
                   acc, src_v, dst_v, rows0, rows1, sem0, sem1):
        c = lax.axis_index("c")
        s = lax.axis_index("s")
        wid = c * NS + s

        pltpu.sync_copy(z_hbm, acc.at[pl.ds(s * rows_per_sub, rows_per_sub)])
        plsc.subcore_barrier()

        bufs = (rows0, rows1)
        sems = (sem0, sem1)
        base = wid * rows_per_worker

        for h in range(2):
            pltpu.sync_copy(src_hbm.at[pl.ds(base + h * half, half)], src_v)
            pltpu.sync_copy(dst_hbm.at[pl.ds(base + h * half, half)], dst_v)
            pltpu.async_copy(x_hbm.at[src_v.at[0]], rows0, sem0)

            def step(g, carry):
                for b in range(2):
                    j = 2 * g + b
                    pltpu.make_async_copy(x_hbm.at[src_v.at[j]], bufs[b],
                                          sems[b]).wait()

                    @pl.when(j + 1 < half)
                    def _():
                        pltpu.async_copy(x_hbm.at[src_v.at[j + 1]],
                                         bufs[1 - b], sems[1 - b])

                    pltpu.sync_copy(bufs[b], acc.at[dst_v.at[j]], add=True)
                return carry

            lax.fori_loop(0, half // 2, step, 0)
        plsc.subcore_barrier()

        pltpu.sync_copy(acc.at[pl.ds(s * rows_per_sub, rows_per_sub)],
                        out_hbm.at[c, pl.ds(s * rows_per_sub, rows_per_sub)])

    return agg_kernel(x_pad, src2d, dst2d, zeros_hbm)


def _tc_combine(partials, x, W_rel, b_rel, W_root):
    n = x.shape[0]
    blk = 80
    grid = n // blk

    def body(p_ref, x_ref, wrel_ref, wroot_ref, b_ref, o_ref):
        agg = p_ref[0] + p_ref[1]
        o_ref[...] = (
            lax.dot_general(agg, wrel_ref[...], (((1,), (1,)), ((), ())),
                            preferred_element_type=jnp.float32)
            + lax.dot_general(x_ref[...], wroot_ref[...], (((1,), (1,)), ((), ())),
                              preferred_element_type=jnp.float32)
            + b_ref[...]
        )

    return pl.pallas_call(
        body,
        grid=(grid,),
        in_specs=[
            pl.BlockSpec((NC, blk, LANE), lambda i: (0, i, 0)),
            pl.BlockSpec((blk, LANE), lambda i: (i, 0)),
            pl.BlockSpec((LANE, LANE), lambda i: (0, 0)),
            pl.BlockSpec((LANE, LANE), lambda i: (0, 0)),
            pl.BlockSpec((1, LANE), lambda i: (0, 0)),
        ],
        out_specs=pl.BlockSpec((blk, LANE), lambda i: (i, 0)),
        out_shape=jax.ShapeDtypeStruct((n, LANE), jnp.float32),
    )(partials, x, W_rel, W_root, b_rel.reshape(1, LANE))


def kernel(x, edge_index, W_rel, b_rel, W_root):
    n, d = x.shape
    e = edge_index.shape[1]
    src = edge_index[0].astype(jnp.int32)
    dst = edge_index[1].astype(jnp.int32)

    per_worker = 16 * LANE * ((e + NW * 16 * LANE - 1) // (NW * 16 * LANE))
    e_pad = NW * per_worker
    src = jnp.concatenate([src, jnp.full((e_pad - e,), n, jnp.int32)])
    dst = jnp.concatenate([dst, jnp.full((e_pad - e,), n, jnp.int32)])
    src2d = src.reshape(-1, LANE)
    dst2d = dst.reshape(-1, LANE)

    x_rows = ((n + 16 + 7) // 8) * 8
    x_pad = jnp.zeros((x_rows, d), jnp.float32).at[:n].set(x)

    acc_rows = ((n + 1 + NS * 8 - 1) // (NS * 8)) * (NS * 8)
    zeros_hbm = jnp.zeros((acc_rows // NS, LANE), jnp.float32)

    partials = _sc_aggregate(x_pad, src2d, dst2d, zeros_hbm,
                             acc_rows, per_worker // LANE)
    return _tc_combine(partials[:, :n], x, W_rel, b_rel, W_root)


# speedup vs baseline: 7.8289x; 2.5636x over previous
import functools

import jax
import jax.numpy as jnp
from jax import lax
from jax.experimental import pallas as pl
from jax.experimental.pallas import tpu as pltpu
from jax.experimental.pallas import tpu_sc as plsc

NC = 2
NS = 16
NW = NC * NS
LANE = 128


def _sc_aggregate(x_pad, src2d, dst2d, zeros_hbm, acc_rows, rows_per_worker):
    rows_per_sub = acc_rows // NS
    half = rows_per_worker // 2
    mesh = plsc.VectorSubcoreMesh(core_axis_name="c", subcore_axis_name="s")

    @functools.partial(
        pl.kernel,
        mesh=mesh,
        out_type=jax.ShapeDtypeStruct((NC, acc_rows, LANE), jnp.float32),
        scratch_types=[
            pltpu.VMEM_SHARED((acc_rows, LANE), jnp.float32),
            pltpu.VMEM((half, LANE), jnp.int32),
            pltpu.VMEM((half, LANE), jnp.int32),
            pltpu.VMEM((LANE, LANE), jnp.float32),
            pltpu.VMEM((LANE, LANE), jnp.float32),
            pltpu.SemaphoreType.DMA,
            pltpu.SemaphoreType.DMA,
        ],
    )
    def agg_kernel(x_hbm, src_hbm, dst_hbm, z_hbm, out_hbm,
                   acc, src_v, dst_v, rows0, rows1, sem0, sem1):
        c = lax.axis_index("c")
        s = lax.axis_index("s")
        wid = c * NS + s

        pltpu.sync_copy(z_hbm, acc.at[pl.ds(s * rows_per_sub, rows_per_sub)])
        plsc.subcore_barrier()

        bufs = (rows0, rows1)
        sems = (sem0, sem1)
        base = wid * rows_per_worker

        for h in range(2):
            pltpu.sync_copy(src_hbm.at[pl.ds(base + h * half, half)], src_v)
            pltpu.sync_copy(dst_hbm.at[pl.ds(base + h * half, half)], dst_v)
            pltpu.async_copy(x_hbm.at[src_v.at[0]], rows0, sem0)

            def step(g, carry):
                for b in range(2):
                    j = 2 * g + b
                    pltpu.make_async_copy(x_hbm.at[src_v.at[j]], bufs[b],
                                          sems[b]).wait()

                    @pl.when(j + 1 < half)
                    def _():
                        pltpu.async_copy(x_hbm.at[src_v.at[j + 1]],
                                         bufs[1 - b], sems[1 - b])

                    pltpu.sync_copy(bufs[b], acc.at[dst_v.at[j]], add=True)
                return carry

            lax.fori_loop(0, half // 2, step, 0)
        plsc.subcore_barrier()

        pltpu.sync_copy(acc.at[pl.ds(s * rows_per_sub, rows_per_sub)],
                        out_hbm.at[c, pl.ds(s * rows_per_sub, rows_per_sub)])

    return agg_kernel(x_pad, src2d, dst2d, zeros_hbm)


def _tc_combine(partials, x, W_rel, b_rel, W_root):
    n = x.shape[0]
    blk = 80
    grid = n // blk

    def body(p_ref, x_ref, wrel_ref, wroot_ref, b_ref, o_ref):
        agg = p_ref[0] + p_ref[1]
        o_ref[...] = (
            lax.dot_general(agg, wrel_ref[...], (((1,), (1,)), ((), ())),
                            preferred_element_type=jnp.float32)
            + lax.dot_general(x_ref[...], wroot_ref[...], (((1,), (1,)), ((), ())),
                              preferred_element_type=jnp.float32)
            + b_ref[...]
        )

    return pl.pallas_call(
        body,
        grid=(grid,),
        in_specs=[
            pl.BlockSpec((NC, blk, LANE), lambda i: (0, i, 0)),
            pl.BlockSpec((blk, LANE), lambda i: (i, 0)),
            pl.BlockSpec((LANE, LANE), lambda i: (0, 0)),
            pl.BlockSpec((LANE, LANE), lambda i: (0, 0)),
            pl.BlockSpec((1, LANE), lambda i: (0, 0)),
        ],
        out_specs=pl.BlockSpec((blk, LANE), lambda i: (i, 0)),
        out_shape=jax.ShapeDtypeStruct((n, LANE), jnp.float32),
    )(partials, x, W_rel, W_root, b_rel.reshape(1, LANE))


def kernel(x, edge_index, W_rel, b_rel, W_root):
    n, d = x.shape
    e = edge_index.shape[1]
    src = edge_index[0].astype(jnp.int32)
    dst = edge_index[1].astype(jnp.int32)

    per_worker = 16 * LANE * ((e + NW * 16 * LANE - 1) // (NW * 16 * LANE))
    e_pad = NW * per_worker
    pad = e_pad - e
    src = jnp.concatenate([src, n + (jnp.arange(pad, dtype=jnp.int32) % 16)])
    dst = jnp.concatenate([dst, jnp.arange(pad, dtype=jnp.int32) % n])
    src2d = src.reshape(-1, LANE)
    dst2d = dst.reshape(-1, LANE)

    x_rows = ((n + 16 + 7) // 8) * 8
    x_pad = jnp.zeros((x_rows, d), jnp.float32).at[:n].set(x)

    acc_rows = ((n + 1 + NS * 8 - 1) // (NS * 8)) * (NS * 8)
    zeros_hbm = jnp.zeros((acc_rows // NS, LANE), jnp.float32)

    partials = _sc_aggregate(x_pad, src2d, dst2d, zeros_hbm,
                             acc_rows, per_worker // LANE)
    return _tc_combine(partials[:, :n], x, W_rel, b_rel, W_root)
